# baseline (speedup 1.0000x reference)
import functools

import jax
import jax.numpy as jnp
from jax import lax
from jax.experimental import pallas as pl
from jax.experimental.pallas import tpu as pltpu
from jax.experimental.pallas import tpu_sc as plsc

IMG = 512.0
C_IN = 39
C_OUT = 31
N = 640000
NW = 32
PER_W = N // NW
CHUNK = 160
NCHUNK = PER_W // CHUNK
GROUPS = CHUNK // 16

_mesh = plsc.VectorSubcoreMesh(core_axis_name="c", subcore_axis_name="s")


@functools.partial(
    pl.kernel,
    mesh=_mesh,
    out_type=jax.ShapeDtypeStruct((N, C_OUT), jnp.float32),
    scratch_types=[
        pltpu.VMEM((2, CHUNK, C_IN), jnp.float32),
        pltpu.VMEM((2, CHUNK, C_OUT), jnp.float32),
        pltpu.SemaphoreType.DMA,
        pltpu.SemaphoreType.DMA,
        pltpu.SemaphoreType.DMA,
        pltpu.SemaphoreType.DMA,
    ],
    compiler_params=pltpu.CompilerParams(
        needs_layout_passes=False, use_tc_tiling_on_sc=True
    ),
)
def _decode_sc(x_hbm, o_hbm, in_v, out_v, in_sem0, in_sem1, wb_sem0, wb_sem1):
    cid = lax.axis_index("c")
    sid = lax.axis_index("s")
    wid = sid * 2 + cid
    base = wid * PER_W
    lanes = lax.iota(jnp.int32, 16)
    in_sems = (in_sem0, in_sem1)
    wb_sems = (wb_sem0, wb_sem1)

    def in_copy(ci, b):
        cbase = base + ci * CHUNK
        return pltpu.make_async_copy(
            x_hbm.at[pl.ds(cbase, CHUNK), :], in_v.at[b], in_sems[b]
        )

    def wb_copy(ci, b):
        cbase = base + ci * CHUNK
        return pltpu.make_async_copy(
            out_v.at[b], o_hbm.at[pl.ds(cbase, CHUNK), :], wb_sems[b]
        )

    def compute(b):
        iv = in_v.at[b]
        ov = out_v.at[b]
        for g in range(GROUPS):
            rows = g * 16 + lanes

            def gat(c):
                col = jnp.full((16,), c, jnp.int32)
                return plsc.load_gather(iv, [rows, col])

            def scat(c, val):
                col = jnp.full((16,), c, jnp.int32)
                plsc.store_scatter(ov, [rows, col], val)

            cx = gat(31)
            cy = gat(32)
            vxw = gat(35) * gat(33)
            vyh = gat(36) * gat(34)
            for i in range(5):
                kx = (gat(21 + 2 * i) * vxw + cx) * IMG
                ky = (gat(22 + 2 * i) * vyh + cy) * IMG
                scat(21 + 2 * i, kx)
                scat(22 + 2 * i, ky)
            for c in range(21):
                scat(c, gat(c))

    in_copy(0, 0).start()
    in_copy(1, 1).start()

    def pair_body(i, carry):
        for b in range(2):
            ci = i * 2 + b

            @pl.when(i > 0)
            def _():
                wb_copy(ci - 2, b).wait()

            in_copy(ci, b).wait()
            compute(b)
            wb_copy(ci, b).start()

            @pl.when(ci + 2 < NCHUNK)
            def _():
                in_copy(ci + 2, b).start()

        return carry

    lax.fori_loop(0, NCHUNK // 2, pair_body, 0)
    wb_copy(NCHUNK - 3, 0).wait()
    in_copy(NCHUNK - 1, 0).wait()
    compute(0)
    wb_copy(NCHUNK - 1, 0).start()
    wb_copy(NCHUNK - 2, 1).wait()
    wb_copy(NCHUNK - 1, 0).wait()


@jax.jit
def kernel(y_pred):
    bt, nb, _ = y_pred.shape
    x = y_pred.reshape(bt * nb, C_IN)
    out = _decode_sc(x)
    return out.reshape(bt, nb, C_OUT)
